# baseline (speedup 1.0000x reference)
import functools

import jax
import jax.numpy as jnp
from jax import lax
from jax.experimental import pallas as pl
from jax.experimental.pallas import tpu as pltpu
from jax.experimental.pallas import tpu_sc as plsc

NUM_GPS = 1000000
EMBED_DIM = 64
BATCH = 16384
HIST = 50

NC = 2
NS = 16
NW = NC * NS
BLK = 128
NBLK_TOTAL = BATCH // BLK * HIST
BPW = NBLK_TOTAL // NW
TB_N = BATCH // BLK
C_SUB = EMBED_DIM // 8
PADC = BLK + 1

NCH_F = 7813
R_PAD = NCH_F * BLK
PADV = 133
FK = 246


def _format_body(tabt_hbm, fmt_hbm, vb0, vb1, pb0, pb1, tb0, tb1,
                 sg0, sg1, ss0, ss1):
    wid = lax.axis_index("s") * NC + lax.axis_index("c")
    sem_g = (sg0, sg1)
    sem_s = (ss0, ss1)
    vbufs = (vb0, vb1)
    pbufs = (pb0, pb1)
    tbufs = (tb0, tb1)

    iota16 = lax.iota(jnp.int32, 16)

    def chunk_of(k):
        return wid + NW * k

    def issue_read(k, b):
        c = chunk_of(k)
        pltpu.async_copy(
            tabt_hbm.at[:, pl.ds(c * BLK, BLK)], vbufs[b], sem_g[b]
        )

    def wait_read(b):
        pltpu.make_async_copy(
            tabt_hbm.at[:, pl.ds(0, BLK)], vbufs[b], sem_g[b]
        ).wait()

    def issue_store(k, b):
        c = chunk_of(k)
        pltpu.async_copy(
            tbufs[b], fmt_hbm.at[pl.ds(c * EMBED_DIM, EMBED_DIM)], sem_s[b]
        )

    def wait_store(b):
        pltpu.make_async_copy(
            tbufs[b], fmt_hbm.at[pl.ds(0, EMBED_DIM)], sem_s[b]
        ).wait()

    def transpose_chunk(b):
        vb = vbufs[b]
        pb = pbufs[b]
        tb_ref = tbufs[b]

        @plsc.parallel_loop(0, EMBED_DIM, step=1, unroll=16)
        def _(c):
            for rg in range(BLK // 16):
                pb[pl.ds(c * PADV + rg * 16, 16)] = vb[c, pl.ds(rg * 16, 16)]

        @plsc.parallel_loop(0, BLK, step=1, unroll=16)
        def _(r):
            for cg in range(EMBED_DIM // 16):
                flat_idx = (iota16 + (16 * cg)) * PADV + r
                v = plsc.load_gather(pb, [flat_idx])
                tb_ref[
                    lax.shift_right_logical(r, 1),
                    pl.ds(lax.bitwise_and(r, 1) * EMBED_DIM + cg * 16, 16),
                ] = v

    issue_read(0, 0)

    @pl.when(chunk_of(1) < NCH_F)
    def _():
        issue_read(1, 1)

    def loop_body(g, _):
        for b in range(2):
            k = 2 * g + b
            valid = chunk_of(k) < NCH_F

            @pl.when(valid)
            def _():
                wait_read(b)

                @pl.when(k >= 2)
                def _():
                    wait_store(b)

                transpose_chunk(b)
                issue_store(k, b)

            @pl.when(chunk_of(k + 2) < NCH_F)
            def _():
                issue_read(k + 2, b)

        return _

    lax.fori_loop(0, FK // 2, loop_body, None)

    for b in range(2):
        last = FK - 2 + b

        @pl.when(chunk_of(last) < NCH_F)
        def _():
            wait_store(b)


def _gather_body(idx_hbm, table_hbm, out_hbm, idx_v, rows_v, tbuf_v,
                 sg0, sg1, ss0, ss1):
    wid = lax.axis_index("s") * NC + lax.axis_index("c")
    sem_g = (sg0, sg1)
    sem_s = (ss0, ss1)

    pltpu.sync_copy(idx_hbm.at[pl.ds(wid * BPW, BPW)], idx_v)

    iota16 = lax.iota(jnp.int32, 16)

    def issue_gather(j, b):
        pltpu.async_copy(table_hbm.at[idx_v.at[j]], rows_v.at[b], sem_g[b])

    def wait_gather(b):
        pltpu.make_async_copy(
            table_hbm.at[pl.ds(0, BLK)], rows_v.at[b], sem_g[b]
        ).wait()

    def transpose_block(b):
        rows = rows_v.at[b]
        tb_ref = tbuf_v.at[b]

        @plsc.parallel_loop(0, BLK, step=1, unroll=16)
        def _(l):
            col_idx = jnp.full((16,), 0, jnp.int32) + l
            for cg in range(EMBED_DIM // 16):
                row_idx = iota16 + (16 * cg)
                v = rows[l, pl.ds(cg * 16, 16)]
                plsc.store_scatter(tb_ref, [row_idx, col_idx], v)

    def issue_stores(j, b):
        g_blk = wid * BPW + j
        h = lax.shift_right_logical(g_blk, 7)
        tb = lax.bitwise_and(g_blk, TB_N - 1)
        for tc in range(C_SUB):
            pltpu.async_copy(
                tbuf_v.at[b].at[pl.ds(tc * 8, 8), pl.ds(0, BLK)],
                out_hbm.at[h, tc, tb],
                sem_s[b],
            )

    def wait_stores(b):
        for tc in range(C_SUB):
            pltpu.make_async_copy(
                tbuf_v.at[b].at[pl.ds(tc * 8, 8), pl.ds(0, BLK)],
                out_hbm.at[0, tc, 0],
                sem_s[b],
            ).wait()

    issue_gather(0, 0)
    issue_gather(1, 1)

    def loop_body(g, _):
        for b in range(2):
            j = 2 * g + b
            wait_gather(b)

            @pl.when(g >= 1)
            def _():
                wait_stores(b)

            transpose_block(b)

            @pl.when(g < BPW // 2 - 1)
            def _():
                issue_gather(j + 2, b)

            issue_stores(j, b)
        return _

    lax.fori_loop(0, BPW // 2, loop_body, None)

    wait_stores(0)
    wait_stores(1)


def kernel(gps_idx, table):
    idx_blocks = gps_idx.astype(jnp.int32).T.reshape(NBLK_TOTAL, BLK)

    mesh = plsc.VectorSubcoreMesh(core_axis_name="c", subcore_axis_name="s")

    fmt = pl.kernel(
        _format_body,
        mesh=mesh,
        out_type=jax.ShapeDtypeStruct((R_PAD // 2, 128), jnp.float32),
        scratch_types=[
            pltpu.VMEM((EMBED_DIM, BLK), jnp.float32),
            pltpu.VMEM((EMBED_DIM, BLK), jnp.float32),
            pltpu.VMEM((EMBED_DIM * PADV,), jnp.float32),
            pltpu.VMEM((EMBED_DIM * PADV,), jnp.float32),
            pltpu.VMEM((EMBED_DIM, BLK), jnp.float32),
            pltpu.VMEM((EMBED_DIM, BLK), jnp.float32),
            pltpu.SemaphoreType.DMA,
            pltpu.SemaphoreType.DMA,
            pltpu.SemaphoreType.DMA,
            pltpu.SemaphoreType.DMA,
        ],
        compiler_params=pltpu.CompilerParams(
            use_tc_tiling_on_sc=True,
            needs_layout_passes=False,
            disable_bounds_checks=True,
        ),
    )(table.T)
    table_rm = fmt.reshape(R_PAD, EMBED_DIM)

    out5 = pl.kernel(
        _gather_body,
        mesh=mesh,
        out_type=jax.ShapeDtypeStruct((HIST, C_SUB, TB_N, 8, BLK), jnp.float32),
        scratch_types=[
            pltpu.VMEM((BPW, BLK), jnp.int32),
            pltpu.VMEM((2, BLK, EMBED_DIM), jnp.float32),
            pltpu.VMEM((2, EMBED_DIM, PADC), jnp.float32),
            pltpu.SemaphoreType.DMA,
            pltpu.SemaphoreType.DMA,
            pltpu.SemaphoreType.DMA,
            pltpu.SemaphoreType.DMA,
        ],
        compiler_params=pltpu.CompilerParams(
            use_tc_tiling_on_sc=False,
            needs_layout_passes=False,
            disable_bounds_checks=True,
        ),
    )(idx_blocks, table_rm)
    return out5.transpose(2, 4, 0, 1, 3).reshape(BATCH, HIST, EMBED_DIM)


# speedup vs baseline: 1.0260x; 1.0260x over previous
import functools

import jax
import jax.numpy as jnp
from jax import lax
from jax.experimental import pallas as pl
from jax.experimental.pallas import tpu as pltpu
from jax.experimental.pallas import tpu_sc as plsc

NUM_GPS = 1000000
EMBED_DIM = 64
BATCH = 16384
HIST = 50

NC = 2
NS = 16
NW = NC * NS
BLK = 128
NBLK_TOTAL = BATCH // BLK * HIST
BPW = NBLK_TOTAL // NW
TB_N = BATCH // BLK
C_SUB = EMBED_DIM // 8
PADC = BLK + 1

NCH_F = 7813
R_PAD = NCH_F * BLK
PADV = 133
FK = 246


def _format_body(tabt_hbm, fmt_hbm, vb0, vb1, pb0, pb1, tb0, tb1,
                 sg0, sg1, ss0, ss1):
    wid = lax.axis_index("s") * NC + lax.axis_index("c")
    sem_g = (sg0, sg1)
    sem_s = (ss0, ss1)
    vbufs = (vb0, vb1)
    pbufs = (pb0, pb1)
    tbufs = (tb0, tb1)

    iota16 = lax.iota(jnp.int32, 16)

    def chunk_of(k):
        return wid + NW * k

    def issue_read(k, b):
        c = chunk_of(k)
        pltpu.async_copy(
            tabt_hbm.at[:, pl.ds(c * BLK, BLK)], vbufs[b], sem_g[b]
        )

    def wait_read(b):
        pltpu.make_async_copy(
            tabt_hbm.at[:, pl.ds(0, BLK)], vbufs[b], sem_g[b]
        ).wait()

    def issue_store(k, b):
        c = chunk_of(k)
        pltpu.async_copy(
            tbufs[b], fmt_hbm.at[pl.ds(c * EMBED_DIM, EMBED_DIM)], sem_s[b]
        )

    def wait_store(b):
        pltpu.make_async_copy(
            tbufs[b], fmt_hbm.at[pl.ds(0, EMBED_DIM)], sem_s[b]
        ).wait()

    def transpose_chunk(b):
        vb = vbufs[b]
        pb = pbufs[b]
        tb_ref = tbufs[b]

        @plsc.parallel_loop(0, EMBED_DIM, step=1, unroll=8)
        def _(c):
            for rg in range(BLK // 16):
                pb[pl.ds(c * PADV + rg * 16, 16)] = vb[c, pl.ds(rg * 16, 16)]

        @plsc.parallel_loop(0, BLK, step=1, unroll=8)
        def _(r):
            for cg in range(EMBED_DIM // 16):
                flat_idx = (iota16 + (16 * cg)) * PADV + r
                v = plsc.load_gather(pb, [flat_idx])
                tb_ref[
                    lax.shift_right_logical(r, 1),
                    pl.ds(lax.bitwise_and(r, 1) * EMBED_DIM + cg * 16, 16),
                ] = v

    issue_read(0, 0)

    @pl.when(chunk_of(1) < NCH_F)
    def _():
        issue_read(1, 1)

    def loop_body(g, _):
        for b in range(2):
            k = 2 * g + b
            valid = chunk_of(k) < NCH_F

            @pl.when(valid)
            def _():
                wait_read(b)

                @pl.when(k >= 2)
                def _():
                    wait_store(b)

                transpose_chunk(b)
                issue_store(k, b)

            @pl.when(chunk_of(k + 2) < NCH_F)
            def _():
                issue_read(k + 2, b)

        return _

    lax.fori_loop(0, FK // 2, loop_body, None)

    for b in range(2):
        last = FK - 2 + b

        @pl.when(chunk_of(last) < NCH_F)
        def _():
            wait_store(b)


def _gather_body(idx_hbm, table_hbm, out_hbm, idx_v, rows_v, tbuf_v,
                 sg0, sg1, ss0, ss1):
    wid = lax.axis_index("s") * NC + lax.axis_index("c")
    sem_g = (sg0, sg1)
    sem_s = (ss0, ss1)

    pltpu.sync_copy(idx_hbm.at[pl.ds(wid * BPW, BPW)], idx_v)

    iota16 = lax.iota(jnp.int32, 16)

    def issue_gather(j, b):
        pltpu.async_copy(table_hbm.at[idx_v.at[j]], rows_v.at[b], sem_g[b])

    def wait_gather(b):
        pltpu.make_async_copy(
            table_hbm.at[pl.ds(0, BLK)], rows_v.at[b], sem_g[b]
        ).wait()

    def transpose_block(b):
        rows = rows_v.at[b]
        tb_ref = tbuf_v.at[b]

        @plsc.parallel_loop(0, BLK, step=1, unroll=8)
        def _(l):
            col_idx = jnp.full((16,), 0, jnp.int32) + l
            for cg in range(EMBED_DIM // 16):
                tc_idx = lax.shift_right_logical(iota16 + (16 * cg), 3)
                sc_idx = lax.bitwise_and(iota16 + (16 * cg), 7)
                v = rows[l, pl.ds(cg * 16, 16)]
                plsc.store_scatter(tb_ref, [tc_idx, sc_idx, col_idx], v)

    def issue_stores(j, b):
        g_blk = wid * BPW + j
        h = lax.shift_right_logical(g_blk, 7)
        tb = lax.bitwise_and(g_blk, TB_N - 1)
        pltpu.async_copy(
            tbuf_v.at[b].at[:, :, pl.ds(0, BLK)],
            out_hbm.at[h, :, tb],
            sem_s[b],
        )

    def wait_stores(b):
        pltpu.make_async_copy(
            tbuf_v.at[b].at[:, :, pl.ds(0, BLK)],
            out_hbm.at[0, :, 0],
            sem_s[b],
        ).wait()

    issue_gather(0, 0)
    issue_gather(1, 1)

    def loop_body(g, _):
        for b in range(2):
            j = 2 * g + b
            wait_gather(b)

            @pl.when(g >= 1)
            def _():
                wait_stores(b)

            transpose_block(b)

            @pl.when(g < BPW // 2 - 1)
            def _():
                issue_gather(j + 2, b)

            issue_stores(j, b)
        return _

    lax.fori_loop(0, BPW // 2, loop_body, None)

    wait_stores(0)
    wait_stores(1)


def kernel(gps_idx, table):
    idx_blocks = gps_idx.astype(jnp.int32).T.reshape(NBLK_TOTAL, BLK)

    mesh = plsc.VectorSubcoreMesh(core_axis_name="c", subcore_axis_name="s")

    fmt = pl.kernel(
        _format_body,
        mesh=mesh,
        out_type=jax.ShapeDtypeStruct((R_PAD // 2, 128), jnp.float32),
        scratch_types=[
            pltpu.VMEM((EMBED_DIM, BLK), jnp.float32),
            pltpu.VMEM((EMBED_DIM, BLK), jnp.float32),
            pltpu.VMEM((EMBED_DIM * PADV,), jnp.float32),
            pltpu.VMEM((EMBED_DIM * PADV,), jnp.float32),
            pltpu.VMEM((EMBED_DIM, BLK), jnp.float32),
            pltpu.VMEM((EMBED_DIM, BLK), jnp.float32),
            pltpu.SemaphoreType.DMA,
            pltpu.SemaphoreType.DMA,
            pltpu.SemaphoreType.DMA,
            pltpu.SemaphoreType.DMA,
        ],
        compiler_params=pltpu.CompilerParams(
            use_tc_tiling_on_sc=True,
            needs_layout_passes=False,
            disable_bounds_checks=True,
        ),
    )(table.T)
    table_rm = fmt.reshape(R_PAD, EMBED_DIM)

    out5 = pl.kernel(
        _gather_body,
        mesh=mesh,
        out_type=jax.ShapeDtypeStruct((HIST, C_SUB, TB_N, 8, BLK), jnp.float32),
        scratch_types=[
            pltpu.VMEM((BPW, BLK), jnp.int32),
            pltpu.VMEM((2, BLK, EMBED_DIM), jnp.float32),
            pltpu.VMEM((2, C_SUB, 8, PADC), jnp.float32),
            pltpu.SemaphoreType.DMA,
            pltpu.SemaphoreType.DMA,
            pltpu.SemaphoreType.DMA,
            pltpu.SemaphoreType.DMA,
        ],
        compiler_params=pltpu.CompilerParams(
            use_tc_tiling_on_sc=False,
            needs_layout_passes=False,
            disable_bounds_checks=True,
        ),
    )(idx_blocks, table_rm)
    return out5.transpose(2, 4, 0, 1, 3).reshape(BATCH, HIST, EMBED_DIM)
